# baseline (speedup 1.0000x reference)
import functools

import jax
import jax.numpy as jnp
from jax import lax
from jax.experimental import pallas as pl
from jax.experimental.pallas import tpu as pltpu
from jax.experimental.pallas import tpu_sc as plsc

N = 10000
M = 5000
E = 320000
D = 128
NUM_LAYERS = 4

NC = 2
NS = 16
NW = NC * NS
EW = E // NW
C = 80
NCH = EW // C
M_PAD = 5120
N_PAD = 10240
CW = 16

_MESH = plsc.VectorSubcoreMesh(core_axis_name="c", subcore_axis_name="s")


def _fill_rows(buf, nrows, value):
    vec = jnp.full((16,), value, jnp.float32)

    @pl.loop(0, nrows)
    def _(r):
        @pl.loop(0, D // 16)
        def _(c16):
            buf[r, pl.ds(c16 * 16, 16)] = vec


def _stage_chunk(dst, src1d, base):
    @pl.loop(0, C // 16)
    def _(j):
        dst[pl.ds(j * 16, 16)] = src1d[pl.ds(base + j * 16, 16)]


_RPT = N_PAD // NS


@functools.partial(
    pl.kernel,
    out_type=jax.ShapeDtypeStruct((NC, N_PAD, D), jnp.float32),
    mesh=_MESH,
    scratch_types=[
        pltpu.VMEM((EW,), jnp.int32),
        pltpu.VMEM((EW,), jnp.int32),
        pltpu.VMEM((C,), jnp.int32),
        pltpu.VMEM((C,), jnp.int32),
        pltpu.VMEM((C, D), jnp.float32),
        pltpu.VMEM((C, D), jnp.float32),
        pltpu.VMEM_SHARED((N_PAD, D), jnp.float32),
        pltpu.SemaphoreType.DMA,
        pltpu.SemaphoreType.DMA,
    ],
)
def _seg_sum(table_hbm, gidx_hbm, sidx_hbm, out_hbm,
             gidx_v, sidx_v, gbuf_a, gbuf_b, rows_a, rows_b, acc,
             sem_a, sem_b):
    cid = lax.axis_index("c")
    sid = lax.axis_index("s")
    wid = cid * NS + sid
    pltpu.sync_copy(gidx_hbm.at[wid], gidx_v)
    pltpu.sync_copy(sidx_hbm.at[wid], sidx_v)
    _fill_rows(rows_a, C, 0.0)
    base = sid * _RPT

    @pl.loop(0, _RPT // C)
    def _(z):
        pltpu.sync_copy(rows_a, acc.at[pl.ds(base + z * C, C)])

    plsc.subcore_barrier()

    def stage_and_start(gbuf, rows, sem, kk):
        _stage_chunk(gbuf, gidx_v, kk * C)
        pltpu.make_async_copy(table_hbm.at[gbuf], rows, sem).start()

    def finish_and_scatter(gbuf, rows, sem, kk):
        pltpu.make_async_copy(table_hbm.at[gbuf], rows, sem).wait()
        _stage_chunk(gbuf, sidx_v, kk * C)
        pltpu.sync_copy(rows, acc.at[gbuf], add=True)

    stage_and_start(gbuf_a, rows_a, sem_a, 0)

    @pl.loop(0, (NCH - 1) // 2)
    def _(i):
        k0 = 2 * i
        stage_and_start(gbuf_b, rows_b, sem_b, k0 + 1)
        finish_and_scatter(gbuf_a, rows_a, sem_a, k0)
        stage_and_start(gbuf_a, rows_a, sem_a, k0 + 2)
        finish_and_scatter(gbuf_b, rows_b, sem_b, k0 + 1)

    finish_and_scatter(gbuf_a, rows_a, sem_a, NCH - 1)

    plsc.subcore_barrier()

    @pl.loop(0, _RPT // C)
    def _(z):
        pltpu.sync_copy(acc.at[pl.ds(base + z * C, C)], rows_a)
        pltpu.sync_copy(rows_a, out_hbm.at[cid, pl.ds(base + z * C, C)])


_RB = 1000


def _enc_body(x_ref, we_ref, be_ref, w0_ref, b0_ref, o_ref):
    t = jnp.dot(x_ref[...], we_ref[...],
                preferred_element_type=jnp.float32) + be_ref[...]
    o_ref[...] = jnp.dot(t, w0_ref[...],
                         preferred_element_type=jnp.float32) + b0_ref[...]


def _encoder(x, W_enc, b_enc, W0, b0):
    return pl.pallas_call(
        _enc_body,
        grid=(N // _RB,),
        in_specs=[
            pl.BlockSpec((_RB, D), lambda i: (i, 0)),
            pl.BlockSpec((D, D), lambda i: (0, 0)),
            pl.BlockSpec((1, D), lambda i: (0, 0)),
            pl.BlockSpec((D, D), lambda i: (0, 0)),
            pl.BlockSpec((1, D), lambda i: (0, 0)),
        ],
        out_specs=pl.BlockSpec((_RB, D), lambda i: (i, 0)),
        out_shape=jax.ShapeDtypeStruct((N_PAD, D), jnp.float32),
    )(x, W_enc, b_enc.reshape(1, D), W0, b0.reshape(1, D))


def _ecomb_body(p_ref, c_ref, o_ref):
    cnt = c_ref[0][:, 0:1] + c_ref[1][:, 0:1]
    inv = 1.0 / jnp.maximum(cnt, 1.0)
    o_ref[...] = (p_ref[0] + p_ref[1]) * inv


def _e_combine(p, cnt_e):
    blk = 1024
    return pl.pallas_call(
        _ecomb_body,
        grid=(M_PAD // blk,),
        in_specs=[
            pl.BlockSpec((NC, blk, D), lambda i: (0, i, 0)),
            pl.BlockSpec((NC, blk, CW), lambda i: (0, i, 0)),
        ],
        out_specs=pl.BlockSpec((blk, D), lambda i: (i, 0)),
        out_shape=jax.ShapeDtypeStruct((N_PAD, D), jnp.float32),
    )(p, cnt_e)


def _layer_norm_relu(h, g, be):
    mu = jnp.mean(h, axis=-1, keepdims=True)
    d = h - mu
    var = jnp.mean(d * d, axis=-1, keepdims=True)
    t = g * d * lax.rsqrt(var + 1e-5) + be
    return jnp.maximum(t, 0.0)


def _make_update_body(first):
    def body(h_ref, q_ref, c_ref, g_ref, be_ref, w_ref, b_ref,
             h_out, x_out):
        cnt = c_ref[0][:, 0:1] + c_ref[1][:, 0:1]
        inv = 1.0 / jnp.maximum(cnt, 1.0)
        r = jnp.maximum((q_ref[0] + q_ref[1]) * inv, 0.0)
        h = r if first else h_ref[...] + r
        h_out[...] = h
        t = _layer_norm_relu(h, g_ref[...], be_ref[...])
        x_out[...] = jnp.dot(t, w_ref[...],
                             preferred_element_type=jnp.float32) + b_ref[...]
    return body


def _layer_update(h, q, cnt_v, g, be, W, b, first):
    return pl.pallas_call(
        _make_update_body(first),
        grid=(N // _RB,),
        in_specs=[
            pl.BlockSpec((_RB, D), lambda i: (i, 0)),
            pl.BlockSpec((NC, _RB, D), lambda i: (0, i, 0)),
            pl.BlockSpec((NC, _RB, CW), lambda i: (0, i, 0)),
            pl.BlockSpec((1, D), lambda i: (0, 0)),
            pl.BlockSpec((1, D), lambda i: (0, 0)),
            pl.BlockSpec((D, D), lambda i: (0, 0)),
            pl.BlockSpec((1, D), lambda i: (0, 0)),
        ],
        out_specs=(pl.BlockSpec((_RB, D), lambda i: (i, 0)),
                   pl.BlockSpec((_RB, D), lambda i: (i, 0))),
        out_shape=(jax.ShapeDtypeStruct((N, D), jnp.float32),
                   jax.ShapeDtypeStruct((N_PAD, D), jnp.float32)),
    )(h, q, cnt_v, g.reshape(1, D), be.reshape(1, D), W, b.reshape(1, D))


def kernel(x, vertex_idx, hyperedge_idx, W_enc, b_enc,
           W0, b0, g0, be0, W1, b1, g1, be1,
           W2, b2, g2, be2, W3, b3, g3, be3,
           W_lin, b_lin):
    gs = [g0, g1, g2, g3]
    bes = [be0, be1, be2, be3]
    Ws = [W0, W1, W2, W3]
    bs = [b0, b1, b2, b3]

    vidx = vertex_idx.astype(jnp.int32).reshape(NW, EW)
    eidx = hyperedge_idx.astype(jnp.int32).reshape(NW, EW)

    ones_table = jnp.ones((N_PAD, D), jnp.float32)
    cnt_e = _seg_sum(ones_table, vidx, eidx)[:, :M_PAD, :CW]
    cnt_v = _seg_sum(ones_table, eidx, vidx)[:, :, :CW]

    xin = _encoder(x, W_enc, b_enc, W0, b0)

    h = None
    for i in range(NUM_LAYERS):
        p = _seg_sum(xin, vidx, eidx)
        e_feat = _e_combine(p, cnt_e)
        q = _seg_sum(e_feat, eidx, vidx)
        if i < NUM_LAYERS - 1:
            g_n, be_n, W_n, b_n = gs[i + 1], bes[i + 1], Ws[i + 1], bs[i + 1]
        else:
            g_n, be_n, W_n, b_n = g0, be0, W_lin, b_lin
        if i == 0:
            h, xin = _layer_update(jnp.zeros((N, D), jnp.float32), q, cnt_v,
                                   g_n, be_n, W_n, b_n, first=True)
        else:
            h, xin = _layer_update(h, q, cnt_v, g_n, be_n, W_n, b_n,
                                   first=False)
    return xin[:N]


# speedup vs baseline: 1.1862x; 1.1862x over previous
import functools

import jax
import jax.numpy as jnp
from jax import lax
from jax.experimental import pallas as pl
from jax.experimental.pallas import tpu as pltpu
from jax.experimental.pallas import tpu_sc as plsc

N = 10000
M = 5000
E = 320000
D = 128
NUM_LAYERS = 4

NC = 2
NS = 16
NW = NC * NS
EW = E // NW
C = 80
NCH = EW // C
M_PAD = 5120
N_PAD = 10240
CW = 16

_MESH = plsc.VectorSubcoreMesh(core_axis_name="c", subcore_axis_name="s")


def _fill_rows(buf, nrows, value):
    vec = jnp.full((16,), value, jnp.float32)

    @pl.loop(0, nrows)
    def _(r):
        @pl.loop(0, D // 16)
        def _(c16):
            buf[r, pl.ds(c16 * 16, 16)] = vec


def _stage_chunk(dst, src1d, base):
    @pl.loop(0, C // 16)
    def _(j):
        dst[pl.ds(j * 16, 16)] = src1d[pl.ds(base + j * 16, 16)]


_RPT = N_PAD // NS
EC = E // NS
NCH2 = EC // C


@functools.partial(
    pl.kernel,
    out_type=jax.ShapeDtypeStruct((NC, N_PAD, CW), jnp.float32),
    mesh=_MESH,
    scratch_types=[
        pltpu.VMEM((EC,), jnp.int32),
        pltpu.VMEM((C,), jnp.int32),
        pltpu.VMEM((C,), jnp.int32),
        pltpu.VMEM((C, CW), jnp.float32),
        pltpu.VMEM((C, CW), jnp.float32),
        pltpu.VMEM_SHARED((N_PAD, CW), jnp.float32),
        pltpu.SemaphoreType.DMA,
        pltpu.SemaphoreType.DMA,
    ],
)
def _seg_counts(idx_hbm, konst_hbm, out_hbm,
                idx_v, cbuf_a, cbuf_b, ones16, zbuf, acc, sem_a, sem_b):
    cid = lax.axis_index("c")
    sid = lax.axis_index("s")
    wid = cid * NS + sid
    pltpu.sync_copy(idx_hbm.at[wid], idx_v)
    pltpu.sync_copy(konst_hbm.at[0], ones16)
    pltpu.sync_copy(konst_hbm.at[1], zbuf)
    base = sid * _RPT

    @pl.loop(0, _RPT // C)
    def _(z):
        pltpu.sync_copy(zbuf, acc.at[pl.ds(base + z * C, C)])

    plsc.subcore_barrier()

    def start(cb, sem, kk):
        _stage_chunk(cb, idx_v, kk * C)
        pltpu.make_async_copy(ones16, acc.at[cb], sem).start(add=True)

    def finish(cb, sem):
        pltpu.make_async_copy(ones16, acc.at[cb], sem).wait()

    start(cbuf_a, sem_a, 0)

    @pl.loop(0, NCH2 // 2 - 1)
    def _(i):
        k0 = 2 * i
        start(cbuf_b, sem_b, k0 + 1)
        finish(cbuf_a, sem_a)
        start(cbuf_a, sem_a, k0 + 2)
        finish(cbuf_b, sem_b)

    start(cbuf_b, sem_b, NCH2 - 1)
    finish(cbuf_a, sem_a)
    finish(cbuf_b, sem_b)

    plsc.subcore_barrier()

    @pl.loop(0, _RPT // C)
    def _(z):
        pltpu.sync_copy(acc.at[pl.ds(base + z * C, C)], zbuf)
        pltpu.sync_copy(zbuf, out_hbm.at[cid, pl.ds(base + z * C, C)])


@functools.partial(
    pl.kernel,
    out_type=jax.ShapeDtypeStruct((NC, N_PAD, D), jnp.float32),
    mesh=_MESH,
    scratch_types=[
        pltpu.VMEM((EW,), jnp.int32),
        pltpu.VMEM((EW,), jnp.int32),
        pltpu.VMEM((C,), jnp.int32),
        pltpu.VMEM((C,), jnp.int32),
        pltpu.VMEM((C, D), jnp.float32),
        pltpu.VMEM((C, D), jnp.float32),
        pltpu.VMEM_SHARED((N_PAD, D), jnp.float32),
        pltpu.SemaphoreType.DMA,
        pltpu.SemaphoreType.DMA,
    ],
)
def _seg_sum(table_hbm, gidx_hbm, sidx_hbm, out_hbm,
             gidx_v, sidx_v, gbuf_a, gbuf_b, rows_a, rows_b, acc,
             sem_a, sem_b):
    cid = lax.axis_index("c")
    sid = lax.axis_index("s")
    wid = cid * NS + sid
    pltpu.sync_copy(gidx_hbm.at[wid], gidx_v)
    pltpu.sync_copy(sidx_hbm.at[wid], sidx_v)
    _fill_rows(rows_a, C, 0.0)
    base = sid * _RPT

    @pl.loop(0, _RPT // C)
    def _(z):
        pltpu.sync_copy(rows_a, acc.at[pl.ds(base + z * C, C)])

    plsc.subcore_barrier()

    def stage_and_start(gbuf, rows, sem, kk):
        _stage_chunk(gbuf, gidx_v, kk * C)
        pltpu.make_async_copy(table_hbm.at[gbuf], rows, sem).start()

    def finish_and_scatter(gbuf, rows, sem, kk):
        pltpu.make_async_copy(table_hbm.at[gbuf], rows, sem).wait()
        _stage_chunk(gbuf, sidx_v, kk * C)
        pltpu.sync_copy(rows, acc.at[gbuf], add=True)

    stage_and_start(gbuf_a, rows_a, sem_a, 0)

    @pl.loop(0, (NCH - 1) // 2)
    def _(i):
        k0 = 2 * i
        stage_and_start(gbuf_b, rows_b, sem_b, k0 + 1)
        finish_and_scatter(gbuf_a, rows_a, sem_a, k0)
        stage_and_start(gbuf_a, rows_a, sem_a, k0 + 2)
        finish_and_scatter(gbuf_b, rows_b, sem_b, k0 + 1)

    finish_and_scatter(gbuf_a, rows_a, sem_a, NCH - 1)

    plsc.subcore_barrier()

    @pl.loop(0, _RPT // C)
    def _(z):
        pltpu.sync_copy(acc.at[pl.ds(base + z * C, C)], rows_a)
        pltpu.sync_copy(rows_a, out_hbm.at[cid, pl.ds(base + z * C, C)])


_RB = 1000


def _enc_body(x_ref, we_ref, be_ref, w0_ref, b0_ref, o_ref):
    t = jnp.dot(x_ref[...], we_ref[...],
                preferred_element_type=jnp.float32) + be_ref[...]
    o_ref[...] = jnp.dot(t, w0_ref[...],
                         preferred_element_type=jnp.float32) + b0_ref[...]


def _encoder(x, W_enc, b_enc, W0, b0):
    return pl.pallas_call(
        _enc_body,
        grid=(N // _RB,),
        in_specs=[
            pl.BlockSpec((_RB, D), lambda i: (i, 0)),
            pl.BlockSpec((D, D), lambda i: (0, 0)),
            pl.BlockSpec((1, D), lambda i: (0, 0)),
            pl.BlockSpec((D, D), lambda i: (0, 0)),
            pl.BlockSpec((1, D), lambda i: (0, 0)),
        ],
        out_specs=pl.BlockSpec((_RB, D), lambda i: (i, 0)),
        out_shape=jax.ShapeDtypeStruct((N_PAD, D), jnp.float32),
    )(x, W_enc, b_enc.reshape(1, D), W0, b0.reshape(1, D))


def _ecomb_body(p_ref, c_ref, o_ref):
    cnt = c_ref[:, 0:1]
    inv = 1.0 / jnp.maximum(cnt, 1.0)
    o_ref[...] = (p_ref[0] + p_ref[1]) * inv


def _e_combine(p, cnt_e):
    blk = 1024
    return pl.pallas_call(
        _ecomb_body,
        grid=(M_PAD // blk,),
        in_specs=[
            pl.BlockSpec((NC, blk, D), lambda i: (0, i, 0)),
            pl.BlockSpec((blk, CW), lambda i: (i, 0)),
        ],
        out_specs=pl.BlockSpec((blk, D), lambda i: (i, 0)),
        out_shape=jax.ShapeDtypeStruct((N_PAD, D), jnp.float32),
    )(p, cnt_e)


def _layer_norm_relu(h, g, be):
    mu = jnp.mean(h, axis=-1, keepdims=True)
    d = h - mu
    var = jnp.mean(d * d, axis=-1, keepdims=True)
    t = g * d * lax.rsqrt(var + 1e-5) + be
    return jnp.maximum(t, 0.0)


def _make_update_body(first):
    def body(h_ref, q_ref, c_ref, g_ref, be_ref, w_ref, b_ref,
             h_out, x_out):
        cnt = c_ref[:, 0:1]
        inv = 1.0 / jnp.maximum(cnt, 1.0)
        r = jnp.maximum((q_ref[0] + q_ref[1]) * inv, 0.0)
        h = r if first else h_ref[...] + r
        h_out[...] = h
        t = _layer_norm_relu(h, g_ref[...], be_ref[...])
        x_out[...] = jnp.dot(t, w_ref[...],
                             preferred_element_type=jnp.float32) + b_ref[...]
    return body


def _layer_update(h, q, cnt_v, g, be, W, b, first):
    return pl.pallas_call(
        _make_update_body(first),
        grid=(N // _RB,),
        in_specs=[
            pl.BlockSpec((_RB, D), lambda i: (i, 0)),
            pl.BlockSpec((NC, _RB, D), lambda i: (0, i, 0)),
            pl.BlockSpec((_RB, CW), lambda i: (i, 0)),
            pl.BlockSpec((1, D), lambda i: (0, 0)),
            pl.BlockSpec((1, D), lambda i: (0, 0)),
            pl.BlockSpec((D, D), lambda i: (0, 0)),
            pl.BlockSpec((1, D), lambda i: (0, 0)),
        ],
        out_specs=(pl.BlockSpec((_RB, D), lambda i: (i, 0)),
                   pl.BlockSpec((_RB, D), lambda i: (i, 0))),
        out_shape=(jax.ShapeDtypeStruct((N, D), jnp.float32),
                   jax.ShapeDtypeStruct((N_PAD, D), jnp.float32)),
    )(h, q, cnt_v, g.reshape(1, D), be.reshape(1, D), W, b.reshape(1, D))


def kernel(x, vertex_idx, hyperedge_idx, W_enc, b_enc,
           W0, b0, g0, be0, W1, b1, g1, be1,
           W2, b2, g2, be2, W3, b3, g3, be3,
           W_lin, b_lin):
    gs = [g0, g1, g2, g3]
    bes = [be0, be1, be2, be3]
    Ws = [W0, W1, W2, W3]
    bs = [b0, b1, b2, b3]

    vflat = vertex_idx.astype(jnp.int32)
    eflat = hyperedge_idx.astype(jnp.int32)
    vidx = vflat.reshape(NW, EW)
    eidx = eflat.reshape(NW, EW)

    idx2 = jnp.concatenate(
        [eflat.reshape(NS, EC), vflat.reshape(NS, EC)], axis=0)
    konst = jnp.stack([jnp.ones((C, CW), jnp.float32),
                       jnp.zeros((C, CW), jnp.float32)])
    cnts = _seg_counts(idx2, konst)
    cnt_e = cnts[0, :M_PAD]
    cnt_v = cnts[1]

    xin = _encoder(x, W_enc, b_enc, W0, b0)

    h = None
    for i in range(NUM_LAYERS):
        p = _seg_sum(xin, vidx, eidx)
        e_feat = _e_combine(p, cnt_e)
        q = _seg_sum(e_feat, eidx, vidx)
        if i < NUM_LAYERS - 1:
            g_n, be_n, W_n, b_n = gs[i + 1], bes[i + 1], Ws[i + 1], bs[i + 1]
        else:
            g_n, be_n, W_n, b_n = g0, be0, W_lin, b_lin
        if i == 0:
            h, xin = _layer_update(jnp.zeros((N, D), jnp.float32), q, cnt_v,
                                   g_n, be_n, W_n, b_n, first=True)
        else:
            h, xin = _layer_update(h, q, cnt_v, g_n, be_n, W_n, b_n,
                                   first=False)
    return xin[:N]
